# Initial kernel scaffold; baseline (speedup 1.0000x reference)
import jax
import jax.numpy as jnp
from jax.experimental import pallas as pl


def kernel(x, edge_index, W1, b1, W2, b2):
    raise NotImplementedError("write your pallas kernel here")


# speedup vs baseline: 20.5563x; 20.5563x over previous
import functools

import jax
import jax.numpy as jnp
from jax import lax
from jax.experimental import pallas as pl
from jax.experimental.pallas import tpu as pltpu
from jax.experimental.pallas import tpu_sc as plsc

NC = 2
NS = 16
NW = NC * NS
L = 16
CHUNK = 128


def _ceil_to(a, m):
    return ((a + m - 1) // m) * m


def _sc_degree(dst_p, npad, epad):
    ept = epad // NW
    sl = npad // NS
    mesh = plsc.VectorSubcoreMesh(core_axis_name="c", subcore_axis_name="s")

    @functools.partial(
        pl.kernel,
        out_type=jax.ShapeDtypeStruct((NC * npad,), jnp.float32),
        mesh=mesh,
        compiler_params=pltpu.CompilerParams(use_tc_tiling_on_sc=False),
        scratch_types=[
            pltpu.VMEM((CHUNK,), jnp.int32),
            pltpu.VMEM((CHUNK,), jnp.float32),
            pltpu.VMEM((sl,), jnp.float32),
            pltpu.VMEM_SHARED((npad,), jnp.float32),
        ],
    )
    def deg_kernel(dst_hbm, out_hbm, idx_v, ones_v, slice_v, deg_sh):
        c = lax.axis_index("c")
        s = lax.axis_index("s")
        wid = c * NS + s

        def zrow(i, _):
            slice_v[pl.ds(i * L, L)] = jnp.zeros((L,), jnp.float32)
            return ()

        lax.fori_loop(0, sl // L, zrow, ())
        for i in range(CHUNK // L):
            ones_v[pl.ds(i * L, L)] = jnp.ones((L,), jnp.float32)
        pltpu.sync_copy(slice_v, deg_sh.at[pl.ds(s * sl, sl)])
        plsc.subcore_barrier()

        base = wid * ept

        def body(g, _):
            off = base + g * CHUNK
            pltpu.sync_copy(dst_hbm.at[pl.ds(off, CHUNK)], idx_v)
            pltpu.sync_copy(ones_v, deg_sh.at[idx_v], add=True)
            return ()

        lax.fori_loop(0, ept // CHUNK, body, ())
        plsc.subcore_barrier()
        pltpu.sync_copy(deg_sh.at[pl.ds(s * sl, sl)], slice_v)
        out_off = pl.multiple_of(c * npad + s * sl, 8)
        pltpu.sync_copy(slice_v, out_hbm.at[pl.ds(out_off, sl)])

    return deg_kernel(dst_p).reshape(NC, npad)


def _sc_aggregate(g_tab, src_p, dst_p, npad, epad):
    ept = epad // NW
    sl = npad // NS
    mesh = plsc.VectorSubcoreMesh(core_axis_name="c", subcore_axis_name="s")

    @functools.partial(
        pl.kernel,
        out_type=jax.ShapeDtypeStruct((NC, npad, L), jnp.float32),
        mesh=mesh,
        compiler_params=pltpu.CompilerParams(use_tc_tiling_on_sc=False),
        scratch_types=[
            pltpu.VMEM((CHUNK,), jnp.int32),
            pltpu.VMEM((CHUNK,), jnp.int32),
            pltpu.VMEM((CHUNK, L), jnp.float32),
            pltpu.VMEM((sl, L), jnp.float32),
            pltpu.VMEM_SHARED((npad, L), jnp.float32),
            pltpu.SemaphoreType.DMA,
        ],
    )
    def agg_kernel(g_hbm, src_hbm, dst_hbm, out_hbm,
                   sidx, didx, rows, slice_v, acc_sh, sem):
        c = lax.axis_index("c")
        s = lax.axis_index("s")
        wid = c * NS + s

        def zrow(i, _):
            slice_v[i, :] = jnp.zeros((L,), jnp.float32)
            return ()

        lax.fori_loop(0, sl, zrow, ())
        pltpu.sync_copy(slice_v, acc_sh.at[pl.ds(s * sl, sl)])
        plsc.subcore_barrier()

        base = wid * ept

        def body(g, _):
            off = base + g * CHUNK
            pltpu.sync_copy(src_hbm.at[pl.ds(off, CHUNK)], sidx)
            pltpu.sync_copy(dst_hbm.at[pl.ds(off, CHUNK)], didx)
            pltpu.async_copy(g_hbm.at[sidx], rows, sem).wait()
            pltpu.sync_copy(rows, acc_sh.at[didx], add=True)
            return ()

        lax.fori_loop(0, ept // CHUNK, body, ())
        plsc.subcore_barrier()
        pltpu.sync_copy(acc_sh.at[pl.ds(s * sl, sl)], slice_v)
        pltpu.sync_copy(slice_v, out_hbm.at[c, pl.ds(s * sl, sl)])

    return agg_kernel(g_tab, src_p, dst_p)


def _tc_k1(x_p, W1, deg_p, blk):
    npad, d = x_p.shape
    h = W1.shape[1]
    grid = npad // blk

    def body(x_ref, w_ref, d0_ref, d1_ref, dis_ref, g1_ref):
        deg = d0_ref[...] + d1_ref[...] + 1.0
        dis = lax.rsqrt(deg)
        hh = jnp.dot(x_ref[...], w_ref[...],
                     preferred_element_type=jnp.float32)
        dis_ref[...] = dis
        g1_ref[...] = dis * hh

    return pl.pallas_call(
        body,
        grid=(grid,),
        in_specs=[
            pl.BlockSpec((blk, d), lambda i: (i, 0)),
            pl.BlockSpec((d, h), lambda i: (0, 0)),
            pl.BlockSpec((blk, 1), lambda i: (i, 0)),
            pl.BlockSpec((blk, 1), lambda i: (i, 0)),
        ],
        out_specs=[
            pl.BlockSpec((blk, 1), lambda i: (i, 0)),
            pl.BlockSpec((blk, h), lambda i: (i, 0)),
        ],
        out_shape=[
            jax.ShapeDtypeStruct((npad, 1), jnp.float32),
            jax.ShapeDtypeStruct((npad, h), jnp.float32),
        ],
    )(x_p, W1, deg_p[0][:, None], deg_p[1][:, None])


def _tc_k2(acc, g1, dis, b1, W2, blk):
    npad, h = g1.shape
    c2 = W2.shape[1]
    grid = npad // blk

    def body(a0_ref, a1_ref, g1_ref, dis_ref, b_ref, w_ref, g2_ref):
        ssum = a0_ref[0] + a1_ref[0] + g1_ref[...]
        dis = dis_ref[...]
        z = jnp.maximum(dis * ssum + b_ref[...], 0.0)
        hh = jnp.dot(z, w_ref[...], preferred_element_type=jnp.float32)
        g2_ref[...] = dis * hh

    return pl.pallas_call(
        body,
        grid=(grid,),
        in_specs=[
            pl.BlockSpec((1, blk, h), lambda i: (0, i, 0)),
            pl.BlockSpec((1, blk, h), lambda i: (1, i, 0)),
            pl.BlockSpec((blk, h), lambda i: (i, 0)),
            pl.BlockSpec((blk, 1), lambda i: (i, 0)),
            pl.BlockSpec((1, h), lambda i: (0, 0)),
            pl.BlockSpec((h, c2), lambda i: (0, 0)),
        ],
        out_specs=pl.BlockSpec((blk, c2), lambda i: (i, 0)),
        out_shape=jax.ShapeDtypeStruct((npad, c2), jnp.float32),
    )(acc, acc, g1, dis, b1[None, :], W2)


def _tc_k3(acc, g2, dis, b2, blk):
    npad, c2 = g2.shape
    grid = npad // blk

    def body(a0_ref, a1_ref, g2_ref, dis_ref, b_ref, out_ref):
        ssum = a0_ref[0] + a1_ref[0] + g2_ref[...]
        o = dis_ref[...] * ssum + b_ref[...]
        m = jnp.max(o, axis=-1, keepdims=True)
        lse = jnp.log(jnp.sum(jnp.exp(o - m), axis=-1, keepdims=True)) + m
        out_ref[...] = o - lse

    return pl.pallas_call(
        body,
        grid=(grid,),
        in_specs=[
            pl.BlockSpec((1, blk, c2), lambda i: (0, i, 0)),
            pl.BlockSpec((1, blk, c2), lambda i: (1, i, 0)),
            pl.BlockSpec((blk, c2), lambda i: (i, 0)),
            pl.BlockSpec((blk, 1), lambda i: (i, 0)),
            pl.BlockSpec((1, c2), lambda i: (0, 0)),
        ],
        out_specs=pl.BlockSpec((blk, c2), lambda i: (i, 0)),
        out_shape=jax.ShapeDtypeStruct((npad, c2), jnp.float32),
    )(acc, acc, g2, dis, b2[None, :])


def kernel(x, edge_index, W1, b1, W2, b2):
    n, d = x.shape
    e = edge_index.shape[1]

    npad = _ceil_to(n + 1, 14 * 128)
    epad = _ceil_to(e, NW * CHUNK)

    src_p = jnp.concatenate(
        [edge_index[0], jnp.zeros((epad - e,), edge_index.dtype)])
    dst_p = jnp.concatenate(
        [edge_index[1], jnp.full((epad - e,), n, edge_index.dtype)])
    x_p = jnp.pad(x, ((0, npad - n), (0, 0)))

    blk = npad // 14

    deg_p = _sc_degree(dst_p, npad, epad)
    dis, g1 = _tc_k1(x_p, W1, deg_p, blk)
    acc1 = _sc_aggregate(g1, src_p, dst_p, npad, epad)
    g2 = _tc_k2(acc1, g1, dis, b1, W2, blk)
    acc2 = _sc_aggregate(g2, src_p, dst_p, npad, epad)
    out = _tc_k3(acc2, g2, dis, b2, blk)
    return out[:n]
